# baseline (speedup 1.0000x reference)
import functools

import jax
import jax.numpy as jnp
from jax import lax
from jax.experimental import pallas as pl
from jax.experimental.pallas import tpu as pltpu
from jax.experimental.pallas import tpu_sc as plsc

_L = 16
_D = 256


def _compute_rows(y_ref, c_ref, w_ref, y_v, c_v, rows_v):
    for i in range(3):
        pltpu.sync_copy(y_ref.at[1 + i], y_v.at[pl.ds(i * _D, _D)])
    pltpu.sync_copy(c_ref, c_v.at[pl.ds(0, 21)])
    cv = c_v[pl.ds(16, _L)]
    c17 = jnp.full((_L,), cv[1], jnp.float32)
    c18 = jnp.full((_L,), cv[2], jnp.float32)
    c19 = jnp.full((_L,), cv[3], jnp.float32)
    row0 = c19 * c17
    row1 = c18 / c19
    for j in range(_D // _L):
        o = j * _L
        rows_v[pl.ds(o, _L)] = row0
        rows_v[pl.ds(_D + o, _L)] = row1
        rows_v[pl.ds(2 * _D + o, _L)] = (
            y_v[pl.ds(2 * _D + o, _L)]
            + y_v[pl.ds(o, _L)]
            + 2.0 * y_v[pl.ds(_D + o, _L)]
        )
    for i in range(3):
        pltpu.sync_copy(rows_v.at[pl.ds(i * _D, _D)], w_ref.at[i])


def _update_body(y_ref, c_ref, w_ref, y_v, c_v, rows_v):
    cid = lax.axis_index("c")
    sid = lax.axis_index("s")

    @pl.when(jnp.logical_and(cid == 0, sid == 0))
    def _():
        _compute_rows(y_ref, c_ref, w_ref, y_v, c_v, rows_v)


@functools.lru_cache(maxsize=None)
def _make_update():
    return pl.kernel(
        _update_body,
        out_type=(),
        mesh=plsc.VectorSubcoreMesh(
            core_axis_name="c", subcore_axis_name="s",
            num_cores=2, num_subcores=16,
        ),
        scratch_types=[
            pltpu.VMEM((3 * _D,), jnp.float32),
            pltpu.VMEM((32,), jnp.float32),
            pltpu.VMEM((3 * _D,), jnp.float32),
        ],
    )


def kernel(y, w, c, t):
    del t
    w_ref = jax.new_ref(w)
    _make_update()(y, c, w_ref)
    return jax.freeze(w_ref)


# speedup vs baseline: 1.4801x; 1.4801x over previous
import jax
import jax.numpy as jnp
from jax.experimental import pallas as pl
from jax.experimental.pallas import tpu as pltpu

_ROWS = 65536
_D = 256
_CH = 16384
_NCH = _ROWS // _CH
_NBUF = 3


def _ring_body(y_ref, c_ref, w_ref, out_ref, buf0, buf1, buf2, yv,
               in_sems, out_sems, ysem):
    bufs = [buf0, buf1, buf2]

    def in_copy(k):
        return pltpu.make_async_copy(
            w_ref.at[pl.ds(k * _CH, _CH)], bufs[k % _NBUF],
            in_sems.at[k % _NBUF])

    def out_copy(k):
        return pltpu.make_async_copy(
            bufs[k % _NBUF], out_ref.at[pl.ds(k * _CH, _CH)],
            out_sems.at[k % _NBUF])

    ycp = pltpu.make_async_copy(y_ref.at[pl.ds(1, 3)], yv, ysem)
    ycp.start()
    for k in range(_NBUF):
        in_copy(k).start()
    ycp.wait()

    for k in range(_NCH):
        in_copy(k).wait()
        if k == 0:
            c17 = c_ref[17]
            c18 = c_ref[18]
            c19 = c_ref[19]
            buf0[0:1, :] = jnp.full((1, _D), c19 * c17, jnp.float32)
            buf0[1:2, :] = jnp.full((1, _D), c18 / c19, jnp.float32)
            buf0[2:3, :] = yv[2:3, :] + yv[0:1, :] + 2.0 * yv[1:2, :]
        out_copy(k).start()
        if k + _NBUF < _NCH:
            out_copy(k).wait()
            in_copy(k + _NBUF).start()
    for k in range(max(0, _NCH - _NBUF), _NCH):
        out_copy(k).wait()


def kernel(y, w, c, t):
    del t
    return pl.pallas_call(
        _ring_body,
        out_shape=jax.ShapeDtypeStruct((_ROWS, _D), jnp.float32),
        in_specs=[
            pl.BlockSpec(memory_space=pl.ANY),
            pl.BlockSpec(memory_space=pltpu.SMEM),
            pl.BlockSpec(memory_space=pl.ANY),
        ],
        out_specs=pl.BlockSpec(memory_space=pl.ANY),
        scratch_shapes=[
            pltpu.VMEM((_CH, _D), jnp.float32),
            pltpu.VMEM((_CH, _D), jnp.float32),
            pltpu.VMEM((_CH, _D), jnp.float32),
            pltpu.VMEM((3, _D), jnp.float32),
            pltpu.SemaphoreType.DMA((_NBUF,)),
            pltpu.SemaphoreType.DMA((_NBUF,)),
            pltpu.SemaphoreType.DMA,
        ],
        compiler_params=pltpu.CompilerParams(
            vmem_limit_bytes=134217728,
        ),
    )(y, c, w)
